# baseline (speedup 1.0000x reference)
import functools

import jax
import jax.numpy as jnp
from jax import lax
from jax.experimental import pallas as pl
from jax.experimental.pallas import tpu as pltpu
from jax.experimental.pallas import tpu_sc as plsc

N_FACTORS = 32
BATCH = 16384
N_ROWS = 1000000
GR_PER_K = N_ROWS // 8
NC, NS, L = 2, 16, 16
NW = NC * NS
BPW = BATCH // NW
CHUNK = 128
NCHUNK = BPW // CHUNK
GPC = CHUNK // L


def _gmfb_body(user_hbm, item_hbm, wb_hbm, ut8, it8, out_hbm,
               idx_uv, idx_iv, off_u, off_i, idxl_u, idxl_i,
               u_g, i_g, out_v, w_v, sem):
    cid = lax.axis_index("c")
    sid = lax.axis_index("s")
    wid = sid * NC + cid
    base = wid * BPW

    pltpu.sync_copy(user_hbm.at[wid], idx_uv)
    pltpu.sync_copy(item_hbm.at[wid], idx_iv)
    pltpu.sync_copy(wb_hbm, w_v)

    def prep_body(v, carry):
        ru = idx_uv[0, pl.ds(v * L, L)]
        ri = idx_iv[0, pl.ds(v * L, L)]
        off_u[0, pl.ds(v * L, L)] = jnp.bitwise_and(ru, 7)
        off_i[0, pl.ds(v * L, L)] = jnp.bitwise_and(ri, 7)
        bu = jnp.right_shift(ru, 3)
        bi = jnp.right_shift(ri, 3)
        c = v // (CHUNK // L)
        jcol = v % (CHUNK // L)
        for k in range(N_FACTORS):
            idxl_u[k, c, pl.ds(jcol * L, L)] = bu + (k * GR_PER_K)
            idxl_i[k, c, pl.ds(jcol * L, L)] = bi + (k * GR_PER_K)
        return carry

    lax.fori_loop(0, BPW // L, prep_body, 0, unroll=False)

    wvec = [w_v[pl.ds(16 * t, L)] for t in range(6)]
    bvec = w_v[pl.ds(96, L)]
    lane = lax.iota(jnp.int32, L)

    def chunk_body(c, carry):
        RING = 8
        copies = []
        for k in range(N_FACTORS):
            copies.append(pltpu.make_async_copy(
                ut8.at[idxl_u.at[k, c]], u_g.at[k], sem))
            copies.append(pltpu.make_async_copy(
                it8.at[idxl_i.at[k, c]], i_g.at[k], sem))
        for n, cp in enumerate(copies):
            cp.start()
            if n >= RING - 1:
                copies[n - (RING - 1)].wait()
        for cp in copies[-(RING - 1):]:
            cp.wait()

        def grp_body(jg, carry2):
            offu = off_u[0, pl.ds(c * CHUNK + jg * L, L)]
            offi = off_i[0, pl.ds(c * CHUNK + jg * L, L)]
            jrow = jg * L + lane
            acc = bvec
            for k in range(N_FACTORS):
                kvec = jnp.full((L,), k, jnp.int32)
                uk = plsc.load_gather(u_g, [kvec, jrow, offu])
                ik = plsc.load_gather(i_g, [kvec, jrow, offi])
                w0k = wvec[k // 16][k % 16]
                w1k = wvec[2 + k // 16][k % 16]
                w2k = wvec[4 + k // 16][k % 16]
                acc = acc + uk * ik * w0k + uk * w1k + ik * w2k
            y = 1.0 / (1.0 + jnp.exp(-acc))
            out_v[pl.ds(c * CHUNK + jg * L, L)] = y
            return carry2

        lax.fori_loop(0, GPC, grp_body, 0)
        return carry

    lax.fori_loop(0, NCHUNK, chunk_body, 0)

    pltpu.sync_copy(out_v, out_hbm.at[pl.ds(base, BPW)])


@jax.jit
def _gmfb(user_r, item_r, wb, ut8, it8):
    mesh = plsc.VectorSubcoreMesh(core_axis_name="c", subcore_axis_name="s",
                                  num_cores=NC, num_subcores=NS)
    f = pl.kernel(
        _gmfb_body,
        out_type=jax.ShapeDtypeStruct((BATCH,), jnp.float32),
        mesh=mesh,
        scratch_types=[
            pltpu.VMEM((1, BPW), jnp.int32),
            pltpu.VMEM((1, BPW), jnp.int32),
            pltpu.VMEM((1, BPW), jnp.int32),
            pltpu.VMEM((1, BPW), jnp.int32),
            pltpu.VMEM((N_FACTORS, NCHUNK, CHUNK), jnp.int32),
            pltpu.VMEM((N_FACTORS, NCHUNK, CHUNK), jnp.int32),
            pltpu.VMEM((N_FACTORS, CHUNK, 8), jnp.float32),
            pltpu.VMEM((N_FACTORS, CHUNK, 8), jnp.float32),
            pltpu.VMEM((BPW,), jnp.float32),
            pltpu.VMEM((112,), jnp.float32),
            pltpu.SemaphoreType.DMA,
        ],
        compiler_params=pltpu.CompilerParams(needs_layout_passes=False,
                                             use_tc_tiling_on_sc=False),
    )
    return f(user_r, item_r, wb, ut8, it8)


def kernel(user, item, user_emb, item_emb, h_w, h_b):
    user_r = user.reshape(NW, 1, BPW)
    item_r = item.reshape(NW, 1, BPW)
    wb = jnp.concatenate([h_w.reshape(N_FACTORS * 3),
                          jnp.broadcast_to(h_b.reshape(1), (L,))])
    ut8 = user_emb.T.reshape(N_FACTORS * GR_PER_K, 8)
    it8 = item_emb.T.reshape(N_FACTORS * GR_PER_K, 8)
    return _gmfb(user_r, item_r, wb, ut8, it8)


# speedup vs baseline: 20.2001x; 20.2001x over previous
import functools

import jax
import jax.numpy as jnp
from jax import lax
from jax.experimental import pallas as pl
from jax.experimental.pallas import tpu as pltpu
from jax.experimental.pallas import tpu_sc as plsc

N_FACTORS = 32
BATCH = 16384
N_ROWS = 1000000
R_TILE = 128
N_RPAD = 1000064
NTILE_R = N_RPAD // R_TILE
NC, NS, L = 2, 16, 16
NW = NC * NS
BPW = BATCH // NW
CHUNK = 128
NCHUNK = BPW // CHUNK
GPC = CHUNK // L


def _gmfb_body(user_hbm, item_hbm, wb_hbm, ut8, it8, out_hbm,
               idx_uv, idx_iv, off_u, off_i, idxl_u, idxl_i,
               u_g, i_g, out_v, w_v, sem):
    cid = lax.axis_index("c")
    sid = lax.axis_index("s")
    wid = sid * NC + cid
    base = wid * BPW

    pltpu.sync_copy(user_hbm.at[wid], idx_uv)
    pltpu.sync_copy(item_hbm.at[wid], idx_iv)
    pltpu.sync_copy(wb_hbm, w_v)

    def prep_body(v, carry):
        ru = idx_uv[0, pl.ds(v * L, L)]
        ri = idx_iv[0, pl.ds(v * L, L)]
        off_u[0, pl.ds(v * L, L)] = jnp.bitwise_and(ru, 7)
        off_i[0, pl.ds(v * L, L)] = jnp.bitwise_and(ri, 7)
        bu = jnp.right_shift(ru, 3) + 112 * jnp.right_shift(ru, 7)
        bi = jnp.right_shift(ri, 3) + 112 * jnp.right_shift(ri, 7)
        c = v // (CHUNK // L)
        jcol = v % (CHUNK // L)
        for k in range(N_FACTORS):
            const_k = (k // 8) * (NTILE_R * R_TILE) + (k % 8) * 16
            idxl_u[k, c, pl.ds(jcol * L, L)] = bu + const_k
            idxl_i[k, c, pl.ds(jcol * L, L)] = bi + const_k
        return carry

    lax.fori_loop(0, BPW // L, prep_body, 0, unroll=False)

    wvec = [w_v[pl.ds(16 * t, L)] for t in range(6)]
    bvec = w_v[pl.ds(96, L)]
    lane = lax.iota(jnp.int32, L)

    def chunk_body(c, carry):
        RING = 8
        copies = []
        for k in range(N_FACTORS):
            copies.append(pltpu.make_async_copy(
                ut8.at[idxl_u.at[k, c]], u_g.at[k], sem))
            copies.append(pltpu.make_async_copy(
                it8.at[idxl_i.at[k, c]], i_g.at[k], sem))
        for n, cp in enumerate(copies):
            cp.start()
            if n >= RING - 1:
                copies[n - (RING - 1)].wait()
        for cp in copies[-(RING - 1):]:
            cp.wait()

        def grp_body(jg, carry2):
            offu = off_u[0, pl.ds(c * CHUNK + jg * L, L)]
            offi = off_i[0, pl.ds(c * CHUNK + jg * L, L)]
            jrow = jg * L + lane
            acc = bvec
            for k in range(N_FACTORS):
                kvec = jnp.full((L,), k, jnp.int32)
                uk = plsc.load_gather(u_g, [kvec, jrow, offu])
                ik = plsc.load_gather(i_g, [kvec, jrow, offi])
                w0k = wvec[k // 16][k % 16]
                w1k = wvec[2 + k // 16][k % 16]
                w2k = wvec[4 + k // 16][k % 16]
                acc = acc + uk * ik * w0k + uk * w1k + ik * w2k
            y = 1.0 / (1.0 + jnp.exp(-acc))
            out_v[pl.ds(c * CHUNK + jg * L, L)] = y
            return carry2

        lax.fori_loop(0, GPC, grp_body, 0)
        return carry

    lax.fori_loop(0, NCHUNK, chunk_body, 0)

    pltpu.sync_copy(out_v, out_hbm.at[pl.ds(base, BPW)])


@jax.jit
def _gmfb(user_r, item_r, wb, ut8, it8):
    mesh = plsc.VectorSubcoreMesh(core_axis_name="c", subcore_axis_name="s",
                                  num_cores=NC, num_subcores=NS)
    f = pl.kernel(
        _gmfb_body,
        out_type=jax.ShapeDtypeStruct((BATCH,), jnp.float32),
        mesh=mesh,
        scratch_types=[
            pltpu.VMEM((1, BPW), jnp.int32),
            pltpu.VMEM((1, BPW), jnp.int32),
            pltpu.VMEM((1, BPW), jnp.int32),
            pltpu.VMEM((1, BPW), jnp.int32),
            pltpu.VMEM((N_FACTORS, NCHUNK, CHUNK), jnp.int32),
            pltpu.VMEM((N_FACTORS, NCHUNK, CHUNK), jnp.int32),
            pltpu.VMEM((N_FACTORS, CHUNK, 8), jnp.float32),
            pltpu.VMEM((N_FACTORS, CHUNK, 8), jnp.float32),
            pltpu.VMEM((BPW,), jnp.float32),
            pltpu.VMEM((112,), jnp.float32),
            pltpu.SemaphoreType.DMA,
        ],
        compiler_params=pltpu.CompilerParams(needs_layout_passes=False,
                                             use_tc_tiling_on_sc=False),
    )
    return f(user_r, item_r, wb, ut8, it8)


def kernel(user, item, user_emb, item_emb, h_w, h_b):
    user_r = user.reshape(NW, 1, BPW)
    item_r = item.reshape(NW, 1, BPW)
    wb = jnp.concatenate([h_w.reshape(N_FACTORS * 3),
                          jnp.broadcast_to(h_b.reshape(1), (L,))])
    def granule_view(emb):
        pe = jnp.pad(emb, ((0, N_RPAD - N_ROWS), (0, 0)))
        v4 = pe.T.reshape(4, 8, NTILE_R, R_TILE).transpose(0, 2, 1, 3)
        return v4.reshape(N_FACTORS * N_RPAD // 8, 8)

    return _gmfb(user_r, item_r, wb, granule_view(user_emb),
                 granule_view(item_emb))


# speedup vs baseline: 21.8998x; 1.0841x over previous
import functools

import jax
import jax.numpy as jnp
from jax import lax
from jax.experimental import pallas as pl
from jax.experimental.pallas import tpu as pltpu
from jax.experimental.pallas import tpu_sc as plsc

N_FACTORS = 32
BATCH = 16384
N_ROWS = 1000000
R_TILE = 128
N_RPAD = 1000064
NTILE_R = N_RPAD // R_TILE
NC, NS, L = 2, 16, 16
NW = NC * NS
BPW = BATCH // NW
CHUNK = 128
NCHUNK = BPW // CHUNK
GPC = CHUNK // L


def _gmfb_body(user_hbm, item_hbm, wb_hbm, ut8, it8, out_hbm,
               idx_uv, idx_iv, off_u, off_i, idxl_u, idxl_i,
               u_g, i_g, out_v, w_v, sem):
    cid = lax.axis_index("c")
    sid = lax.axis_index("s")
    wid = sid * NC + cid
    base = wid * BPW

    pltpu.sync_copy(user_hbm.at[wid], idx_uv)
    pltpu.sync_copy(item_hbm.at[wid], idx_iv)
    pltpu.sync_copy(wb_hbm, w_v)

    def prep_body(v, carry):
        ru = idx_uv[0, pl.ds(v * L, L)]
        ri = idx_iv[0, pl.ds(v * L, L)]
        off_u[0, pl.ds(v * L, L)] = jnp.bitwise_and(ru, 7)
        off_i[0, pl.ds(v * L, L)] = jnp.bitwise_and(ri, 7)
        bu = jnp.right_shift(ru, 3) + 112 * jnp.right_shift(ru, 7)
        bi = jnp.right_shift(ri, 3) + 112 * jnp.right_shift(ri, 7)
        c = v // (CHUNK // L)
        jcol = v % (CHUNK // L)
        for k in range(N_FACTORS):
            const_k = (k // 8) * (NTILE_R * R_TILE) + (k % 8) * 16
            idxl_u[k, c, pl.ds(jcol * L, L)] = bu + const_k
            idxl_i[k, c, pl.ds(jcol * L, L)] = bi + const_k
        return carry

    lax.fori_loop(0, BPW // L, prep_body, 0, unroll=False)

    wvec = [w_v[pl.ds(16 * t, L)] for t in range(6)]
    bvec = w_v[pl.ds(96, L)]
    lane = lax.iota(jnp.int32, L)

    def chunk_body(c, carry):
        RING = 16
        copies = []
        for k in range(N_FACTORS):
            copies.append(pltpu.make_async_copy(
                ut8.at[idxl_u.at[k, c]], u_g.at[k], sem))
            copies.append(pltpu.make_async_copy(
                it8.at[idxl_i.at[k, c]], i_g.at[k], sem))
        for n, cp in enumerate(copies):
            cp.start()
            if n >= RING - 1:
                copies[n - (RING - 1)].wait()
        for cp in copies[-(RING - 1):]:
            cp.wait()

        def grp_body(jg, carry2):
            offu = off_u[0, pl.ds(c * CHUNK + jg * L, L)]
            offi = off_i[0, pl.ds(c * CHUNK + jg * L, L)]
            jrow = jg * L + lane
            acc = bvec
            for k in range(N_FACTORS):
                kvec = jnp.full((L,), k, jnp.int32)
                uk = plsc.load_gather(u_g, [kvec, jrow, offu])
                ik = plsc.load_gather(i_g, [kvec, jrow, offi])
                w0k = wvec[k // 16][k % 16]
                w1k = wvec[2 + k // 16][k % 16]
                w2k = wvec[4 + k // 16][k % 16]
                acc = acc + uk * ik * w0k + uk * w1k + ik * w2k
            y = 1.0 / (1.0 + jnp.exp(-acc))
            out_v[pl.ds(c * CHUNK + jg * L, L)] = y
            return carry2

        lax.fori_loop(0, GPC, grp_body, 0)
        return carry

    lax.fori_loop(0, NCHUNK, chunk_body, 0)

    pltpu.sync_copy(out_v, out_hbm.at[pl.ds(base, BPW)])


@jax.jit
def _gmfb(user_r, item_r, wb, ut8, it8):
    mesh = plsc.VectorSubcoreMesh(core_axis_name="c", subcore_axis_name="s",
                                  num_cores=NC, num_subcores=NS)
    f = pl.kernel(
        _gmfb_body,
        out_type=jax.ShapeDtypeStruct((BATCH,), jnp.float32),
        mesh=mesh,
        scratch_types=[
            pltpu.VMEM((1, BPW), jnp.int32),
            pltpu.VMEM((1, BPW), jnp.int32),
            pltpu.VMEM((1, BPW), jnp.int32),
            pltpu.VMEM((1, BPW), jnp.int32),
            pltpu.VMEM((N_FACTORS, NCHUNK, CHUNK), jnp.int32),
            pltpu.VMEM((N_FACTORS, NCHUNK, CHUNK), jnp.int32),
            pltpu.VMEM((N_FACTORS, CHUNK, 8), jnp.float32),
            pltpu.VMEM((N_FACTORS, CHUNK, 8), jnp.float32),
            pltpu.VMEM((BPW,), jnp.float32),
            pltpu.VMEM((112,), jnp.float32),
            pltpu.SemaphoreType.DMA,
        ],
        compiler_params=pltpu.CompilerParams(needs_layout_passes=False,
                                             use_tc_tiling_on_sc=False),
    )
    return f(user_r, item_r, wb, ut8, it8)


def kernel(user, item, user_emb, item_emb, h_w, h_b):
    user_r = user.reshape(NW, 1, BPW)
    item_r = item.reshape(NW, 1, BPW)
    wb = jnp.concatenate([h_w.reshape(N_FACTORS * 3),
                          jnp.broadcast_to(h_b.reshape(1), (L,))])
    def granule_view(emb):
        pe = jnp.pad(emb, ((0, N_RPAD - N_ROWS), (0, 0)))
        v4 = pe.T.reshape(4, 8, NTILE_R, R_TILE).transpose(0, 2, 1, 3)
        return v4.reshape(N_FACTORS * N_RPAD // 8, 8)

    return _gmfb(user_r, item_r, wb, granule_view(user_emb),
                 granule_view(item_emb))


# speedup vs baseline: 22.9968x; 1.0501x over previous
import functools

import jax
import jax.numpy as jnp
from jax import lax
from jax.experimental import pallas as pl
from jax.experimental.pallas import tpu as pltpu
from jax.experimental.pallas import tpu_sc as plsc

N_FACTORS = 32
BATCH = 16384
N_ROWS = 1000000
R_TILE = 128
N_RPAD = 1000064
NTILE_R = N_RPAD // R_TILE
NC, NS, L = 2, 16, 16
NW = NC * NS
BPW = BATCH // NW
CHUNK = 128
NCHUNK = BPW // CHUNK
GPC = CHUNK // L


def _gmfb_body(user_hbm, item_hbm, wb_hbm, ut8, it8, out_hbm,
               idx_uv, idx_iv, off_u, off_i, idxl_u, idxl_i,
               u_g, i_g, out_v, acc_v, w_v, sem, sem2):
    cid = lax.axis_index("c")
    sid = lax.axis_index("s")
    wid = sid * NC + cid
    base = wid * BPW

    pltpu.sync_copy(user_hbm.at[wid], idx_uv)
    pltpu.sync_copy(item_hbm.at[wid], idx_iv)
    pltpu.sync_copy(wb_hbm, w_v)

    def prep_body(v, carry):
        ru = idx_uv[0, pl.ds(v * L, L)]
        ri = idx_iv[0, pl.ds(v * L, L)]
        off_u[0, pl.ds(v * L, L)] = jnp.bitwise_and(ru, 7)
        off_i[0, pl.ds(v * L, L)] = jnp.bitwise_and(ri, 7)
        bu = jnp.right_shift(ru, 3) + 112 * jnp.right_shift(ru, 7)
        bi = jnp.right_shift(ri, 3) + 112 * jnp.right_shift(ri, 7)
        c = v // (CHUNK // L)
        jcol = v % (CHUNK // L)
        for k in range(N_FACTORS):
            const_k = (k // 8) * (NTILE_R * R_TILE) + (k % 8) * 16
            idxl_u[k, c, pl.ds(jcol * L, L)] = bu + const_k
            idxl_i[k, c, pl.ds(jcol * L, L)] = bi + const_k
        return carry

    lax.fori_loop(0, BPW // L, prep_body, 0, unroll=False)

    wvec = [w_v[pl.ds(16 * t, L)] for t in range(6)]
    bvec = w_v[pl.ds(96, L)]
    lane = lax.iota(jnp.int32, L)

    KB = 8
    NSUB = N_FACTORS // KB
    SUBS = NCHUNK * NSUB
    sems = [sem, sem2]

    def fire_sub(c, kb, s_sem):
        for i in range(KB):
            k = kb * KB + i
            pltpu.make_async_copy(ut8.at[idxl_u.at[k, c]], u_g.at[k],
                                  s_sem).start()
            pltpu.make_async_copy(it8.at[idxl_i.at[k, c]], i_g.at[k],
                                  s_sem).start()

    def drain_sub(s_sem):
        for _ in range(2 * KB):
            pltpu.make_async_copy(ut8.at[idxl_u.at[0, 0]], u_g.at[0],
                                  s_sem).wait()

    def compute_sub(c, kb):
        def grp_body(jg, carry2):
            offu = off_u[0, pl.ds(c * CHUNK + jg * L, L)]
            offi = off_i[0, pl.ds(c * CHUNK + jg * L, L)]
            jrow = jg * L + lane
            acc = bvec if kb == 0 else acc_v[pl.ds(jg * L, L)]
            for i in range(KB):
                k = kb * KB + i
                kvec = jnp.full((L,), k, jnp.int32)
                uk = plsc.load_gather(u_g, [kvec, jrow, offu])
                ik = plsc.load_gather(i_g, [kvec, jrow, offi])
                w0k = wvec[k // 16][k % 16]
                w1k = wvec[2 + k // 16][k % 16]
                w2k = wvec[4 + k // 16][k % 16]
                acc = acc + uk * ik * w0k + uk * w1k + ik * w2k
            if kb == NSUB - 1:
                y = 1.0 / (1.0 + jnp.exp(-acc))
                out_v[pl.ds(c * CHUNK + jg * L, L)] = y
            else:
                acc_v[pl.ds(jg * L, L)] = acc
            return carry2

        lax.fori_loop(0, GPC, grp_body, 0)

    fire_sub(0, 0, sems[0])
    for s in range(1, SUBS + 1):
        if s < SUBS:
            c, kb = divmod(s, NSUB)
            fire_sub(c, kb, sems[s % 2])
        pc, pkb = divmod(s - 1, NSUB)
        drain_sub(sems[(s - 1) % 2])
        compute_sub(pc, pkb)

    pltpu.sync_copy(out_v, out_hbm.at[pl.ds(base, BPW)])


@jax.jit
def _gmfb(user_r, item_r, wb, ut8, it8):
    mesh = plsc.VectorSubcoreMesh(core_axis_name="c", subcore_axis_name="s",
                                  num_cores=NC, num_subcores=NS)
    f = pl.kernel(
        _gmfb_body,
        out_type=jax.ShapeDtypeStruct((BATCH,), jnp.float32),
        mesh=mesh,
        scratch_types=[
            pltpu.VMEM((1, BPW), jnp.int32),
            pltpu.VMEM((1, BPW), jnp.int32),
            pltpu.VMEM((1, BPW), jnp.int32),
            pltpu.VMEM((1, BPW), jnp.int32),
            pltpu.VMEM((N_FACTORS, NCHUNK, CHUNK), jnp.int32),
            pltpu.VMEM((N_FACTORS, NCHUNK, CHUNK), jnp.int32),
            pltpu.VMEM((N_FACTORS, CHUNK, 8), jnp.float32),
            pltpu.VMEM((N_FACTORS, CHUNK, 8), jnp.float32),
            pltpu.VMEM((BPW,), jnp.float32),
            pltpu.VMEM((CHUNK,), jnp.float32),
            pltpu.VMEM((112,), jnp.float32),
            pltpu.SemaphoreType.DMA,
            pltpu.SemaphoreType.DMA,
        ],
        compiler_params=pltpu.CompilerParams(needs_layout_passes=False,
                                             use_tc_tiling_on_sc=False),
    )
    return f(user_r, item_r, wb, ut8, it8)


def kernel(user, item, user_emb, item_emb, h_w, h_b):
    user_r = user.reshape(NW, 1, BPW)
    item_r = item.reshape(NW, 1, BPW)
    wb = jnp.concatenate([h_w.reshape(N_FACTORS * 3),
                          jnp.broadcast_to(h_b.reshape(1), (L,))])
    def granule_view(emb):
        pe = jnp.pad(emb, ((0, N_RPAD - N_ROWS), (0, 0)))
        v4 = pe.T.reshape(4, 8, NTILE_R, R_TILE).transpose(0, 2, 1, 3)
        return v4.reshape(N_FACTORS * N_RPAD // 8, 8)

    return _gmfb(user_r, item_r, wb, granule_view(user_emb),
                 granule_view(item_emb))


# speedup vs baseline: 23.5341x; 1.0234x over previous
import functools

import jax
import jax.numpy as jnp
from jax import lax
from jax.experimental import pallas as pl
from jax.experimental.pallas import tpu as pltpu
from jax.experimental.pallas import tpu_sc as plsc

N_FACTORS = 32
BATCH = 16384
N_ROWS = 1000000
R_TILE = 128
N_RPAD = 1000064
NTILE_R = N_RPAD // R_TILE
NC, NS, L = 2, 16, 16
NW = NC * NS
BPW = BATCH // NW
CHUNK = 128
NCHUNK = BPW // CHUNK
GPC = CHUNK // L
KB = 8
NSUB = N_FACTORS // KB
SUBS = NCHUNK * NSUB


def _prep_indices(idx_v, off, idxl):
    def prep_body(v, carry):
        r = idx_v[0, pl.ds(v * L, L)]
        off[0, pl.ds(v * L, L)] = jnp.bitwise_and(r, 7)
        b = jnp.right_shift(r, 3) + 112 * jnp.right_shift(r, 7)
        c = v // (CHUNK // L)
        jcol = v % (CHUNK // L)
        for k in range(N_FACTORS):
            const_k = (k // 8) * (NTILE_R * R_TILE) + (k % 8) * 16
            idxl[k, c, pl.ds(jcol * L, L)] = b + const_k
        return carry

    lax.fori_loop(0, BPW // L, prep_body, 0, unroll=False)


def _fire_sub(tab, idxl, g_buf, c, kb, s_sem):
    for i in range(KB):
        k = kb * KB + i
        pltpu.make_async_copy(tab.at[idxl.at[k, c]], g_buf.at[k],
                              s_sem).start()


def _drain(tab, idxl, g_buf, s_sem, n):
    for _ in range(n):
        pltpu.make_async_copy(tab.at[idxl.at[0, 0]], g_buf.at[0],
                              s_sem).wait()


def _ugather_body(user_hbm, ut8, ucols_hbm,
                  idx_uv, off_u, idxl_u, u_g, u_cols, sem, sem2):
    cid = lax.axis_index("c")
    sid = lax.axis_index("s")
    wid = sid * NC + cid

    pltpu.sync_copy(user_hbm.at[wid], idx_uv)
    _prep_indices(idx_uv, off_u, idxl_u)
    lane = lax.iota(jnp.int32, L)
    sems = [sem, sem2]

    def extract_sub(c, kb):
        def grp_body(jg, carry2):
            offu = off_u[0, pl.ds(c * CHUNK + jg * L, L)]
            jrow = jg * L + lane
            for i in range(KB):
                k = kb * KB + i
                kvec = jnp.full((L,), k, jnp.int32)
                uk = plsc.load_gather(u_g, [kvec, jrow, offu])
                u_cols[k, pl.ds(c * CHUNK + jg * L, L)] = uk
            return carry2

        lax.fori_loop(0, GPC, grp_body, 0)

    _fire_sub(ut8, idxl_u, u_g, 0, 0, sems[0])
    for s in range(1, SUBS + 1):
        if s < SUBS:
            c, kb = divmod(s, NSUB)
            _fire_sub(ut8, idxl_u, u_g, c, kb, sems[s % 2])
        pc, pkb = divmod(s - 1, NSUB)
        _drain(ut8, idxl_u, u_g, sems[(s - 1) % 2], KB)
        extract_sub(pc, pkb)

    pltpu.sync_copy(u_cols, ucols_hbm.at[wid])


def _ifuse_body(item_hbm, wb_hbm, ucols_hbm, it8, out_hbm,
                idx_iv, off_i, idxl_i, i_g, u_cols, out_v, acc_v,
                w_v, sem, sem2):
    cid = lax.axis_index("c")
    sid = lax.axis_index("s")
    wid = sid * NC + cid
    base = wid * BPW

    pltpu.sync_copy(item_hbm.at[wid], idx_iv)
    pltpu.sync_copy(wb_hbm, w_v)
    pltpu.sync_copy(ucols_hbm.at[wid], u_cols)
    _prep_indices(idx_iv, off_i, idxl_i)

    wvec = [w_v[pl.ds(16 * t, L)] for t in range(6)]
    bvec = w_v[pl.ds(96, L)]
    lane = lax.iota(jnp.int32, L)
    sems = [sem, sem2]

    def compute_sub(c, kb):
        def grp_body(jg, carry2):
            offi = off_i[0, pl.ds(c * CHUNK + jg * L, L)]
            jrow = jg * L + lane
            acc = bvec if kb == 0 else acc_v[pl.ds(jg * L, L)]
            for i in range(KB):
                k = kb * KB + i
                kvec = jnp.full((L,), k, jnp.int32)
                uk = u_cols[k, pl.ds(c * CHUNK + jg * L, L)]
                ik = plsc.load_gather(i_g, [kvec, jrow, offi])
                w0k = wvec[k // 16][k % 16]
                w1k = wvec[2 + k // 16][k % 16]
                w2k = wvec[4 + k // 16][k % 16]
                acc = acc + uk * ik * w0k + uk * w1k + ik * w2k
            if kb == NSUB - 1:
                y = 1.0 / (1.0 + jnp.exp(-acc))
                out_v[pl.ds(c * CHUNK + jg * L, L)] = y
            else:
                acc_v[pl.ds(jg * L, L)] = acc
            return carry2

        lax.fori_loop(0, GPC, grp_body, 0)

    _fire_sub(it8, idxl_i, i_g, 0, 0, sems[0])
    for s in range(1, SUBS + 1):
        if s < SUBS:
            c, kb = divmod(s, NSUB)
            _fire_sub(it8, idxl_i, i_g, c, kb, sems[s % 2])
        pc, pkb = divmod(s - 1, NSUB)
        _drain(it8, idxl_i, i_g, sems[(s - 1) % 2], KB)
        compute_sub(pc, pkb)

    pltpu.sync_copy(out_v, out_hbm.at[pl.ds(base, BPW)])


_SC_PARAMS = pltpu.CompilerParams(needs_layout_passes=False,
                                  use_tc_tiling_on_sc=False)


@jax.jit
def _gmfb(user_r, item_r, wb, ut8, it8):
    mesh = plsc.VectorSubcoreMesh(core_axis_name="c", subcore_axis_name="s",
                                  num_cores=NC, num_subcores=NS)
    ug = pl.kernel(
        _ugather_body,
        out_type=jax.ShapeDtypeStruct((NW, N_FACTORS, BPW), jnp.float32),
        mesh=mesh,
        scratch_types=[
            pltpu.VMEM((1, BPW), jnp.int32),
            pltpu.VMEM((1, BPW), jnp.int32),
            pltpu.VMEM((N_FACTORS, NCHUNK, CHUNK), jnp.int32),
            pltpu.VMEM((N_FACTORS, CHUNK, 8), jnp.float32),
            pltpu.VMEM((N_FACTORS, BPW), jnp.float32),
            pltpu.SemaphoreType.DMA,
            pltpu.SemaphoreType.DMA,
        ],
        compiler_params=_SC_PARAMS,
    )
    ucols = ug(user_r, ut8)

    fi = pl.kernel(
        _ifuse_body,
        out_type=jax.ShapeDtypeStruct((BATCH,), jnp.float32),
        mesh=mesh,
        scratch_types=[
            pltpu.VMEM((1, BPW), jnp.int32),
            pltpu.VMEM((1, BPW), jnp.int32),
            pltpu.VMEM((N_FACTORS, NCHUNK, CHUNK), jnp.int32),
            pltpu.VMEM((N_FACTORS, CHUNK, 8), jnp.float32),
            pltpu.VMEM((N_FACTORS, BPW), jnp.float32),
            pltpu.VMEM((BPW,), jnp.float32),
            pltpu.VMEM((CHUNK,), jnp.float32),
            pltpu.VMEM((112,), jnp.float32),
            pltpu.SemaphoreType.DMA,
            pltpu.SemaphoreType.DMA,
        ],
        compiler_params=_SC_PARAMS,
    )
    return fi(item_r, wb, ucols, it8)


def kernel(user, item, user_emb, item_emb, h_w, h_b):
    user_r = user.reshape(NW, 1, BPW)
    item_r = item.reshape(NW, 1, BPW)
    wb = jnp.concatenate([h_w.reshape(N_FACTORS * 3),
                          jnp.broadcast_to(h_b.reshape(1), (L,))])

    def granule_view(emb):
        pe = jnp.pad(emb, ((0, N_RPAD - N_ROWS), (0, 0)))
        v4 = pe.T.reshape(4, 8, NTILE_R, R_TILE).transpose(0, 2, 1, 3)
        return v4.reshape(N_FACTORS * N_RPAD // 8, 8)

    return _gmfb(user_r, item_r, wb, granule_view(user_emb),
                 granule_view(item_emb))


# speedup vs baseline: 23.7717x; 1.0101x over previous
import functools

import jax
import jax.numpy as jnp
from jax import lax
from jax.experimental import pallas as pl
from jax.experimental.pallas import tpu as pltpu
from jax.experimental.pallas import tpu_sc as plsc

N_FACTORS = 32
BATCH = 16384
N_ROWS = 1000000
R_TILE = 128
N_RPAD = 1000064
NTILE_R = N_RPAD // R_TILE
NC, NS, L = 2, 16, 16
NW = NC * NS
BPW = BATCH // NW
CHUNK = 128
NCHUNK = BPW // CHUNK
GPC = CHUNK // L
KB = 16
NSUB = N_FACTORS // KB
SUBS = NCHUNK * NSUB


def _prep_indices(idx_v, off, idxl):
    def prep_body(v, carry):
        r = idx_v[0, pl.ds(v * L, L)]
        off[0, pl.ds(v * L, L)] = jnp.bitwise_and(r, 7)
        b = jnp.right_shift(r, 3) + 112 * jnp.right_shift(r, 7)
        c = v // (CHUNK // L)
        jcol = v % (CHUNK // L)
        for k in range(N_FACTORS):
            const_k = (k // 8) * (NTILE_R * R_TILE) + (k % 8) * 16
            idxl[k, c, pl.ds(jcol * L, L)] = b + const_k
        return carry

    lax.fori_loop(0, BPW // L, prep_body, 0, unroll=False)


def _fire_sub(tab, idxl, g_buf, c, kb, s_sem):
    for i in range(KB):
        k = kb * KB + i
        pltpu.make_async_copy(tab.at[idxl.at[k, c]], g_buf.at[k],
                              s_sem).start()


def _drain(tab, idxl, g_buf, s_sem, n):
    for _ in range(n):
        pltpu.make_async_copy(tab.at[idxl.at[0, 0]], g_buf.at[0],
                              s_sem).wait()


def _ugather_body(user_hbm, ut8, ucols_hbm,
                  idx_uv, off_u, idxl_u, u_g, u_cols, sem, sem2):
    cid = lax.axis_index("c")
    sid = lax.axis_index("s")
    wid = sid * NC + cid

    pltpu.sync_copy(user_hbm.at[wid], idx_uv)
    _prep_indices(idx_uv, off_u, idxl_u)
    lane = lax.iota(jnp.int32, L)
    sems = [sem, sem2]

    def extract_sub(c, kb):
        def grp_body(jg, carry2):
            offu = off_u[0, pl.ds(c * CHUNK + jg * L, L)]
            jrow = jg * L + lane
            for i in range(KB):
                k = kb * KB + i
                kvec = jnp.full((L,), k, jnp.int32)
                uk = plsc.load_gather(u_g, [kvec, jrow, offu])
                u_cols[k, pl.ds(c * CHUNK + jg * L, L)] = uk
            return carry2

        lax.fori_loop(0, GPC, grp_body, 0)

    _fire_sub(ut8, idxl_u, u_g, 0, 0, sems[0])
    for s in range(1, SUBS + 1):
        if s < SUBS:
            c, kb = divmod(s, NSUB)
            _fire_sub(ut8, idxl_u, u_g, c, kb, sems[s % 2])
        pc, pkb = divmod(s - 1, NSUB)
        _drain(ut8, idxl_u, u_g, sems[(s - 1) % 2], KB)
        extract_sub(pc, pkb)

    pltpu.sync_copy(u_cols, ucols_hbm.at[wid])


def _ifuse_body(item_hbm, wb_hbm, ucols_hbm, it8, out_hbm,
                idx_iv, off_i, idxl_i, i_g, u_cols, out_v, acc_v,
                w_v, sem, sem2):
    cid = lax.axis_index("c")
    sid = lax.axis_index("s")
    wid = sid * NC + cid
    base = wid * BPW

    pltpu.sync_copy(item_hbm.at[wid], idx_iv)
    pltpu.sync_copy(wb_hbm, w_v)
    pltpu.sync_copy(ucols_hbm.at[wid], u_cols)
    _prep_indices(idx_iv, off_i, idxl_i)

    wvec = [w_v[pl.ds(16 * t, L)] for t in range(6)]
    bvec = w_v[pl.ds(96, L)]
    lane = lax.iota(jnp.int32, L)
    sems = [sem, sem2]

    def compute_sub(c, kb):
        def grp_body(jg, carry2):
            offi = off_i[0, pl.ds(c * CHUNK + jg * L, L)]
            jrow = jg * L + lane
            acc = bvec if kb == 0 else acc_v[pl.ds(jg * L, L)]
            for i in range(KB):
                k = kb * KB + i
                kvec = jnp.full((L,), k, jnp.int32)
                uk = u_cols[k, pl.ds(c * CHUNK + jg * L, L)]
                ik = plsc.load_gather(i_g, [kvec, jrow, offi])
                w0k = wvec[k // 16][k % 16]
                w1k = wvec[2 + k // 16][k % 16]
                w2k = wvec[4 + k // 16][k % 16]
                acc = acc + uk * ik * w0k + uk * w1k + ik * w2k
            if kb == NSUB - 1:
                y = 1.0 / (1.0 + jnp.exp(-acc))
                out_v[pl.ds(c * CHUNK + jg * L, L)] = y
            else:
                acc_v[pl.ds(jg * L, L)] = acc
            return carry2

        lax.fori_loop(0, GPC, grp_body, 0)

    _fire_sub(it8, idxl_i, i_g, 0, 0, sems[0])
    for s in range(1, SUBS + 1):
        if s < SUBS:
            c, kb = divmod(s, NSUB)
            _fire_sub(it8, idxl_i, i_g, c, kb, sems[s % 2])
        pc, pkb = divmod(s - 1, NSUB)
        _drain(it8, idxl_i, i_g, sems[(s - 1) % 2], KB)
        compute_sub(pc, pkb)

    pltpu.sync_copy(out_v, out_hbm.at[pl.ds(base, BPW)])


_SC_PARAMS = pltpu.CompilerParams(needs_layout_passes=False,
                                  use_tc_tiling_on_sc=False)


@jax.jit
def _gmfb(user_r, item_r, wb, ut8, it8):
    mesh = plsc.VectorSubcoreMesh(core_axis_name="c", subcore_axis_name="s",
                                  num_cores=NC, num_subcores=NS)
    ug = pl.kernel(
        _ugather_body,
        out_type=jax.ShapeDtypeStruct((NW, N_FACTORS, BPW), jnp.float32),
        mesh=mesh,
        scratch_types=[
            pltpu.VMEM((1, BPW), jnp.int32),
            pltpu.VMEM((1, BPW), jnp.int32),
            pltpu.VMEM((N_FACTORS, NCHUNK, CHUNK), jnp.int32),
            pltpu.VMEM((N_FACTORS, CHUNK, 8), jnp.float32),
            pltpu.VMEM((N_FACTORS, BPW), jnp.float32),
            pltpu.SemaphoreType.DMA,
            pltpu.SemaphoreType.DMA,
        ],
        compiler_params=_SC_PARAMS,
    )
    ucols = ug(user_r, ut8)

    fi = pl.kernel(
        _ifuse_body,
        out_type=jax.ShapeDtypeStruct((BATCH,), jnp.float32),
        mesh=mesh,
        scratch_types=[
            pltpu.VMEM((1, BPW), jnp.int32),
            pltpu.VMEM((1, BPW), jnp.int32),
            pltpu.VMEM((N_FACTORS, NCHUNK, CHUNK), jnp.int32),
            pltpu.VMEM((N_FACTORS, CHUNK, 8), jnp.float32),
            pltpu.VMEM((N_FACTORS, BPW), jnp.float32),
            pltpu.VMEM((BPW,), jnp.float32),
            pltpu.VMEM((CHUNK,), jnp.float32),
            pltpu.VMEM((112,), jnp.float32),
            pltpu.SemaphoreType.DMA,
            pltpu.SemaphoreType.DMA,
        ],
        compiler_params=_SC_PARAMS,
    )
    return fi(item_r, wb, ucols, it8)


def kernel(user, item, user_emb, item_emb, h_w, h_b):
    user_r = user.reshape(NW, 1, BPW)
    item_r = item.reshape(NW, 1, BPW)
    wb = jnp.concatenate([h_w.reshape(N_FACTORS * 3),
                          jnp.broadcast_to(h_b.reshape(1), (L,))])

    def granule_view(emb):
        pe = jnp.pad(emb, ((0, N_RPAD - N_ROWS), (0, 0)))
        v4 = pe.T.reshape(4, 8, NTILE_R, R_TILE).transpose(0, 2, 1, 3)
        return v4.reshape(N_FACTORS * N_RPAD // 8, 8)

    return _gmfb(user_r, item_r, wb, granule_view(user_emb),
                 granule_view(item_emb))
